# baseline (speedup 1.0000x reference)
import functools

import jax
import jax.numpy as jnp
from jax import lax
from jax.experimental import pallas as pl
from jax.experimental.pallas import tpu as pltpu

_B, _S, _D, _H, _DH, _HID = 1, 2048, 2048, 16, 128, 256
_BQ = 256
_BM = 512
_INT_MIN = -2147483648

_HIGH = lax.Precision.HIGHEST


def _projrot_kernel(hs_ref, wq_ref, wk_ref, cos_ref, sin_ref, q_ref, k_ref):
    hs = hs_ref[...]
    c = cos_ref[...]
    s = sin_ref[...]

    def rot(w_ref, o_ref):
        p = jnp.dot(hs, w_ref[...], preferred_element_type=jnp.float32)
        p1 = p[:, :_DH // 2]
        p2 = p[:, _DH // 2:]
        o_ref[0, :, :_DH // 2] = p1 * c - p2 * s
        o_ref[0, :, _DH // 2:] = p2 * c + p1 * s

    rot(wq_ref, q_ref)
    rot(wk_ref, k_ref)


def _head_kernel(q_ref, k_ref, xf_ref, xb_ref, eps_ref, w1_ref, b1_ref,
                 rw_ref, w2_ref, b2_ref, out_ref, keys_ref):
    i = pl.program_id(1)
    q = q_ref[0]
    k = k_ref[0]
    scores = lax.dot_general(q, k, (((1,), (1,)), ((), ())),
                             preferred_element_type=jnp.float32)

    row = i * _BQ + lax.broadcasted_iota(jnp.int32, (_BQ, _S), 0)
    col = lax.broadcasted_iota(jnp.int32, (_BQ, _S), 1)
    causal = col < row
    bits = lax.bitcast_convert_type(scores, jnp.int32)
    keys = jnp.where(bits >= 0, bits, bits ^ jnp.int32(0x7FFFFFFF))
    keys = jnp.where(causal, keys, jnp.int32(_INT_MIN))
    keys_ref[...] = keys

    kcount = jnp.maximum(1, (row[:, :1] + 1) // 2)
    nch = i + 1

    def count_ge(cand):
        def chunk_body(c, acc):
            kc = keys_ref[:, pl.ds(c * _BQ, _BQ)]
            return acc + jnp.sum((kc >= cand).astype(jnp.int32), axis=1,
                                 keepdims=True)
        return lax.fori_loop(0, nch, chunk_body,
                             jnp.zeros((_BQ, 1), jnp.int32))

    def bs_body(it, t):
        bit = jnp.int32(30) - it
        cand = t + (jnp.int32(1) << bit)
        return jnp.where(count_ge(cand) >= kcount, cand, t)

    t0 = jnp.where(count_ge(jnp.zeros((_BQ, 1), jnp.int32)) >= kcount,
                   jnp.int32(0), jnp.int32(_INT_MIN))
    t = lax.fori_loop(0, 31, bs_body, t0)
    adj = ((keys >= t) & causal).astype(jnp.float32)

    x_full = xf_ref[0]
    x_blk = xb_ref[0]
    agg = jnp.dot(adj, x_full, preferred_element_type=jnp.float32,
                  precision=_HIGH)
    agg = agg + eps_ref[0] * x_blk

    h1 = jnp.dot(agg, w1_ref[0], preferred_element_type=jnp.float32,
                 precision=_HIGH) + b1_ref[0]
    rms = jnp.sqrt(jnp.mean(h1 * h1, axis=-1, keepdims=True) + 1e-6)
    h1n = (h1 / rms) * rw_ref[0]
    a = h1n * (1.0 / (1.0 + jnp.exp(-h1n)))
    h2 = jnp.dot(a, w2_ref[0], preferred_element_type=jnp.float32,
                 precision=_HIGH) + b2_ref[0]
    out_ref[...] = h2


def _out_kernel(m_ref, wo_ref, o_ref):
    o_ref[...] = jnp.dot(m_ref[...], wo_ref[...],
                         preferred_element_type=jnp.float32, precision=_HIGH)


@jax.jit
def kernel(hidden_states, Wq, Wk, gin_eps, gin_W1, gin_b1, gin_rms_w,
           gin_W2, gin_b2, Wo):
    hs = hidden_states[0]
    pos = jnp.arange(_S, dtype=jnp.float32)
    inv = 1.0 / (10000.0 ** (jnp.arange(0, _DH, 2, dtype=jnp.float32) / _DH))
    freqs = pos[:, None] * inv[None, :]
    cos = jnp.cos(freqs)
    sin = jnp.sin(freqs)

    q, k = pl.pallas_call(
        _projrot_kernel,
        grid=(_H,),
        in_specs=[
            pl.BlockSpec((_S, _D), lambda h: (0, 0)),
            pl.BlockSpec((_D, _DH), lambda h: (0, h)),
            pl.BlockSpec((_D, _DH), lambda h: (0, h)),
            pl.BlockSpec((_S, _DH // 2), lambda h: (0, 0)),
            pl.BlockSpec((_S, _DH // 2), lambda h: (0, 0)),
        ],
        out_specs=[
            pl.BlockSpec((1, _S, _DH), lambda h: (h, 0, 0)),
            pl.BlockSpec((1, _S, _DH), lambda h: (h, 0, 0)),
        ],
        out_shape=[
            jax.ShapeDtypeStruct((_H, _S, _DH), jnp.float32),
            jax.ShapeDtypeStruct((_H, _S, _DH), jnp.float32),
        ],
        compiler_params=pltpu.CompilerParams(
            dimension_semantics=("arbitrary",)),
    )(hs, Wq, Wk, cos, sin)

    x3 = hs.reshape(_S, _H, _DH).transpose(1, 0, 2)
    eps3 = gin_eps.reshape(_H, 1, 1)
    b1r = gin_b1.reshape(_H, 1, _HID)
    rwr = gin_rms_w.reshape(_H, 1, _HID)
    b2r = gin_b2.reshape(_H, 1, _DH)

    merged = pl.pallas_call(
        _head_kernel,
        grid=(_H, _S // _BQ),
        in_specs=[
            pl.BlockSpec((1, _BQ, _DH), lambda h, i: (h, i, 0)),
            pl.BlockSpec((1, _S, _DH), lambda h, i: (h, 0, 0)),
            pl.BlockSpec((1, _S, _DH), lambda h, i: (h, 0, 0)),
            pl.BlockSpec((1, _BQ, _DH), lambda h, i: (h, i, 0)),
            pl.BlockSpec((1, 1, 1), lambda h, i: (h, 0, 0)),
            pl.BlockSpec((1, _DH, _HID), lambda h, i: (h, 0, 0)),
            pl.BlockSpec((1, 1, _HID), lambda h, i: (h, 0, 0)),
            pl.BlockSpec((1, 1, _HID), lambda h, i: (h, 0, 0)),
            pl.BlockSpec((1, _HID, _DH), lambda h, i: (h, 0, 0)),
            pl.BlockSpec((1, 1, _DH), lambda h, i: (h, 0, 0)),
        ],
        out_specs=pl.BlockSpec((_BQ, _DH), lambda h, i: (i, h)),
        out_shape=jax.ShapeDtypeStruct((_S, _H * _DH), jnp.float32),
        scratch_shapes=[pltpu.VMEM((_BQ, _S), jnp.int32)],
        compiler_params=pltpu.CompilerParams(
            dimension_semantics=("arbitrary", "arbitrary")),
    )(q, k, x3, x3, eps3, gin_W1, b1r, rwr, gin_W2, b2r)

    out = pl.pallas_call(
        _out_kernel,
        grid=(_S // _BM,),
        in_specs=[
            pl.BlockSpec((_BM, _H * _DH), lambda m: (m, 0)),
            pl.BlockSpec((_H * _DH, _D), lambda m: (0, 0)),
        ],
        out_specs=pl.BlockSpec((_BM, _D), lambda m: (m, 0)),
        out_shape=jax.ShapeDtypeStruct((_S, _D), jnp.float32),
        compiler_params=pltpu.CompilerParams(
            dimension_semantics=("arbitrary",)),
    )(merged, Wo)

    return out[None]


# speedup vs baseline: 2.0554x; 2.0554x over previous
import functools

import jax
import jax.numpy as jnp
from jax import lax
from jax.experimental import pallas as pl
from jax.experimental.pallas import tpu as pltpu

_B, _S, _D, _H, _DH, _HID = 1, 2048, 2048, 16, 128, 256
_BQ = 256
_BM = 512
_INT_MIN = -2147483648

_HIGH = lax.Precision.HIGHEST


def _projrot_kernel(hs_ref, wq_ref, wk_ref, cos_ref, sin_ref, q_ref, k_ref):
    hs = hs_ref[...]
    c = cos_ref[...]
    s = sin_ref[...]

    def rot(w_ref, o_ref):
        p = jnp.dot(hs, w_ref[...], preferred_element_type=jnp.float32)
        p1 = p[:, :_DH // 2]
        p2 = p[:, _DH // 2:]
        o_ref[0, :, :_DH // 2] = p1 * c - p2 * s
        o_ref[0, :, _DH // 2:] = p2 * c + p1 * s

    rot(wq_ref, q_ref)
    rot(wk_ref, k_ref)


def _head_kernel(q_ref, k_ref, xf_ref, xb_ref, eps_ref, w1_ref, b1_ref,
                 rw_ref, w2_ref, b2_ref, out_ref, *, width, i_off):
    i = i_off + pl.program_id(1)
    q = q_ref[0]
    k = k_ref[0]
    scores = lax.dot_general(q, k, (((1,), (1,)), ((), ())),
                             preferred_element_type=jnp.float32)

    row = i * _BQ + lax.broadcasted_iota(jnp.int32, (_BQ, width), 0)
    col = lax.broadcasted_iota(jnp.int32, (_BQ, width), 1)
    causal = col < row
    bits = lax.bitcast_convert_type(scores, jnp.int32)
    keys = jnp.where(bits >= 0, bits, bits ^ jnp.int32(0x7FFFFFFF))
    keys = jnp.where(causal, keys, jnp.int32(_INT_MIN))

    kcount = jnp.maximum(1, (row[:, :1] + 1) // 2)

    def bs_body(it, t):
        bit = jnp.int32(30) - it
        cand = t + (jnp.int32(1) << bit)
        cnt = jnp.sum((keys >= cand).astype(jnp.int32), axis=1, keepdims=True)
        return jnp.where(cnt >= kcount, cand, t)

    cnt0 = jnp.sum((keys >= 0).astype(jnp.int32), axis=1, keepdims=True)
    t0 = jnp.where(cnt0 >= kcount, jnp.int32(0), jnp.int32(_INT_MIN))
    t = lax.fori_loop(0, 31, bs_body, t0)
    adj = ((keys >= t) & causal).astype(jnp.float32)

    x_full = xf_ref[0]
    x_blk = xb_ref[0]
    agg = jnp.dot(adj, x_full, preferred_element_type=jnp.float32,
                  precision=_HIGH)
    agg = agg + eps_ref[0] * x_blk

    h1 = jnp.dot(agg, w1_ref[0], preferred_element_type=jnp.float32,
                 precision=_HIGH) + b1_ref[0]
    rms = jnp.sqrt(jnp.mean(h1 * h1, axis=-1, keepdims=True) + 1e-6)
    h1n = (h1 / rms) * rw_ref[0]
    a = h1n * (1.0 / (1.0 + jnp.exp(-h1n)))
    h2 = jnp.dot(a, w2_ref[0], preferred_element_type=jnp.float32,
                 precision=_HIGH) + b2_ref[0]
    out_ref[...] = h2


def _out_kernel(m_ref, wo_ref, o_ref):
    o_ref[...] = jnp.dot(m_ref[...], wo_ref[...],
                         preferred_element_type=jnp.float32, precision=_HIGH)


@jax.jit
def kernel(hidden_states, Wq, Wk, gin_eps, gin_W1, gin_b1, gin_rms_w,
           gin_W2, gin_b2, Wo):
    hs = hidden_states[0]
    pos = jnp.arange(_S, dtype=jnp.float32)
    inv = 1.0 / (10000.0 ** (jnp.arange(0, _DH, 2, dtype=jnp.float32) / _DH))
    freqs = pos[:, None] * inv[None, :]
    cos = jnp.cos(freqs)
    sin = jnp.sin(freqs)

    q, k = pl.pallas_call(
        _projrot_kernel,
        grid=(_H,),
        in_specs=[
            pl.BlockSpec((_S, _D), lambda h: (0, 0)),
            pl.BlockSpec((_D, _DH), lambda h: (0, h)),
            pl.BlockSpec((_D, _DH), lambda h: (0, h)),
            pl.BlockSpec((_S, _DH // 2), lambda h: (0, 0)),
            pl.BlockSpec((_S, _DH // 2), lambda h: (0, 0)),
        ],
        out_specs=[
            pl.BlockSpec((1, _S, _DH), lambda h: (h, 0, 0)),
            pl.BlockSpec((1, _S, _DH), lambda h: (h, 0, 0)),
        ],
        out_shape=[
            jax.ShapeDtypeStruct((_H, _S, _DH), jnp.float32),
            jax.ShapeDtypeStruct((_H, _S, _DH), jnp.float32),
        ],
        compiler_params=pltpu.CompilerParams(
            dimension_semantics=("arbitrary",)),
    )(hs, Wq, Wk, cos, sin)

    x3 = hs.reshape(_S, _H, _DH).transpose(1, 0, 2)
    eps3 = gin_eps.reshape(_H, 1, 1)
    b1r = gin_b1.reshape(_H, 1, _HID)
    rwr = gin_rms_w.reshape(_H, 1, _HID)
    b2r = gin_b2.reshape(_H, 1, _DH)

    def head_call(width, i_off, n_i):
        return pl.pallas_call(
            functools.partial(_head_kernel, width=width, i_off=i_off),
            grid=(_H, n_i),
            in_specs=[
                pl.BlockSpec((1, _BQ, _DH), lambda h, i: (h, i_off + i, 0)),
                pl.BlockSpec((1, width, _DH), lambda h, i: (h, 0, 0)),
                pl.BlockSpec((1, width, _DH), lambda h, i: (h, 0, 0)),
                pl.BlockSpec((1, _BQ, _DH), lambda h, i: (h, i_off + i, 0)),
                pl.BlockSpec((1, 1, 1), lambda h, i: (h, 0, 0)),
                pl.BlockSpec((1, _DH, _HID), lambda h, i: (h, 0, 0)),
                pl.BlockSpec((1, 1, _HID), lambda h, i: (h, 0, 0)),
                pl.BlockSpec((1, 1, _HID), lambda h, i: (h, 0, 0)),
                pl.BlockSpec((1, _HID, _DH), lambda h, i: (h, 0, 0)),
                pl.BlockSpec((1, 1, _DH), lambda h, i: (h, 0, 0)),
            ],
            out_specs=pl.BlockSpec((_BQ, _DH), lambda h, i: (i, h)),
            out_shape=jax.ShapeDtypeStruct((n_i * _BQ, _H * _DH),
                                           jnp.float32),
            compiler_params=pltpu.CompilerParams(
                dimension_semantics=("arbitrary", "arbitrary")),
        )(q, k, x3, x3, eps3, gin_W1, b1r, rwr, gin_W2, b2r)

    n_half = _S // _BQ // 2
    merged = jnp.concatenate(
        [head_call(_S // 2, 0, n_half), head_call(_S, n_half, n_half)],
        axis=0)

    out = pl.pallas_call(
        _out_kernel,
        grid=(_S // _BM,),
        in_specs=[
            pl.BlockSpec((_BM, _H * _DH), lambda m: (m, 0)),
            pl.BlockSpec((_H * _DH, _D), lambda m: (0, 0)),
        ],
        out_specs=pl.BlockSpec((_BM, _D), lambda m: (m, 0)),
        out_shape=jax.ShapeDtypeStruct((_S, _D), jnp.float32),
        compiler_params=pltpu.CompilerParams(
            dimension_semantics=("arbitrary",)),
    )(merged, Wo)

    return out[None]
